# Initial kernel scaffold; baseline (speedup 1.0000x reference)
import functools

import jax
import jax.numpy as jnp
from jax import lax
from jax.experimental import pallas as pl
from jax.experimental.pallas import tpu as pltpu
from jax.experimental.pallas import tpu_sc as plsc

N = 10000
D = 256
H = 512
C = 2 * H
OUT = 256
EPS = 1e-5

NPAD = 10240
RB = 512
NRB = NPAD // RB
FB = 128
CHUNK = 128
NTILES = 16
NCH = 79
SC_ROWS = 10016
RPT = SC_ROWS // NTILES
DUMMY = N
F32 = jnp.float32


def _make_spmm(slots, with_deg):
    per = slots // 2
    mesh = plsc.VectorSubcoreMesh(core_axis_name="c", subcore_axis_name="s")
    out_type = [jax.ShapeDtypeStruct((slots, NPAD, FB), F32)]
    scratch = [
        pltpu.VMEM((NCH, CHUNK), jnp.int32),
        pltpu.VMEM((NCH, CHUNK), jnp.int32),
        pltpu.VMEM((CHUNK, FB), F32),
        pltpu.VMEM((64, FB), F32),
        pltpu.VMEM_SHARED((SC_ROWS, FB), F32),
        pltpu.SemaphoreType.DMA,
    ]
    if with_deg:
        out_type.append(jax.ShapeDtypeStruct((NPAD, 16), F32))
        scratch += [
            pltpu.VMEM((CHUNK, 16), F32),
            pltpu.VMEM((RPT, 16), F32),
            pltpu.VMEM_SHARED((SC_ROWS, 16), F32),
        ]

    @functools.partial(pl.kernel, mesh=mesh, out_type=out_type,
                       scratch_types=scratch)
    def spmm(xflat, idx, dstt, *refs):
        if with_deg:
            agg_out, deg_out = refs[0], refs[1]
            gidx, dstv, rows, zbuf, agg_sp, sem, ones_v, degz, deg_sp = refs[2:]
        else:
            agg_out = refs[0]
            gidx, dstv, rows, zbuf, agg_sp, sem = refs[1:]
        c = lax.axis_index("c")
        t = lax.axis_index("s")
        base_r = t * RPT

        def zrow(i, _):
            for l in range(FB // 16):
                zbuf[i, pl.ds(l * 16, 16)] = jnp.zeros((16,), F32)
            return 0
        lax.fori_loop(0, 64, zrow, 0)

        pltpu.sync_copy(dstt.at[pl.ds(t * NCH, NCH)], dstv)

        for k in range(per):
            slot = c * per + k
            pltpu.sync_copy(idx.at[slot, pl.ds(t * NCH, NCH)], gidx)
            for z in range(9):
                pltpu.sync_copy(zbuf, agg_sp.at[pl.ds(base_r + z * 64, 64)])
            pltpu.sync_copy(zbuf.at[pl.ds(0, RPT - 576)],
                            agg_sp.at[pl.ds(base_r + 576, RPT - 576)])
            plsc.subcore_barrier()

            def chunk_body(j, _):
                pltpu.async_copy(xflat.at[gidx.at[j]], rows, sem).wait()
                pltpu.sync_copy(rows, agg_sp.at[dstv.at[j]], add=True)
                return 0
            lax.fori_loop(0, NCH, chunk_body, 0)
            plsc.subcore_barrier()
            pltpu.sync_copy(agg_sp.at[pl.ds(base_r, RPT)],
                            agg_out.at[slot, pl.ds(base_r, RPT)])

        if with_deg:
            def orow(i, _):
                ones_v[i, pl.ds(0, 16)] = jnp.ones((16,), F32)
                return 0
            lax.fori_loop(0, CHUNK, orow, 0)

            def dzrow(i, _):
                degz[i, pl.ds(0, 16)] = jnp.zeros((16,), F32)
                return 0
            lax.fori_loop(0, RPT, dzrow, 0)

            @pl.when(c == 0)
            def _deg():
                pltpu.sync_copy(degz, deg_sp.at[pl.ds(base_r, RPT)])
                plsc.subcore_barrier()

                def dchunk(j, _):
                    pltpu.sync_copy(ones_v, deg_sp.at[dstv.at[j]], add=True)
                    return 0
                lax.fori_loop(0, NCH, dchunk, 0)
                plsc.subcore_barrier()
                pltpu.sync_copy(deg_sp.at[pl.ds(base_r, RPT)],
                                deg_out.at[pl.ds(base_r, RPT)])

    return spmm


_spmm4 = _make_spmm(4, True)
_spmm8 = _make_spmm(8, False)
_spmm2 = _make_spmm(2, False)


def _inv_deg(deg_ref):
    return 1.0 / jnp.maximum(deg_ref[:, :1], 1.0)


def _k0_body(agg_ref, xs_ref, deg_ref, wl_ref, wr_ref, b_ref, out_ref):
    inv = _inv_deg(deg_ref)
    wl = wl_ref[0]
    acc = jnp.dot(agg_ref[0] * inv, wl[:FB], preferred_element_type=F32)
    acc += jnp.dot(agg_ref[1] * inv, wl[FB:], preferred_element_type=F32)
    acc += jnp.dot(xs_ref[0], wr_ref[0], preferred_element_type=F32)
    out_ref[...] = jnp.maximum(acc + b_ref[0][None, :], 0.0)


def _k1_body(x_ref, agg_ref, deg_ref, wl1_ref, wr1_ref, b1_ref,
             wl2_ref, wr2_ref, b2_ref, y_ref, r2_ref):
    inv = _inv_deg(deg_ref)
    acc = jnp.dot(x_ref[...], wr1_ref[...], preferred_element_type=F32)
    for fb in range(C // FB):
        acc += jnp.dot(agg_ref[fb] * inv, wl1_ref[fb * FB:(fb + 1) * FB],
                       preferred_element_type=F32)
    xn = jnp.maximum(acc + b1_ref[0][None, :], 0.0)
    y_ref[...] = jnp.dot(xn, wl2_ref[...], preferred_element_type=F32)
    r2_ref[...] = (jnp.dot(xn, wr2_ref[...], preferred_element_type=F32)
                   + b2_ref[0][None, :])


def _k2_body(aggy_ref, r2_ref, deg_ref, out_ref):
    inv = _inv_deg(deg_ref)
    out_ref[...] = (jnp.concatenate([aggy_ref[0] * inv, aggy_ref[1] * inv],
                                    axis=1) + r2_ref[...])


def _conv0(agg0, xs, deg, wl0, wr0, b0):
    return pl.pallas_call(
        _k0_body,
        grid=(2, NRB),
        in_specs=[
            pl.BlockSpec((2, RB, FB), lambda s, r: (s, r, 0)),
            pl.BlockSpec((1, RB, D), lambda s, r: (s, r, 0)),
            pl.BlockSpec((RB, 16), lambda s, r: (r, 0)),
            pl.BlockSpec((1, D, H), lambda s, r: (s, 0, 0)),
            pl.BlockSpec((1, D, H), lambda s, r: (s, 0, 0)),
            pl.BlockSpec((1, H), lambda s, r: (s, 0)),
        ],
        out_specs=pl.BlockSpec((RB, H), lambda s, r: (r, s)),
        out_shape=jax.ShapeDtypeStruct((NPAD, C), F32),
    )(agg0, xs, deg, wl0, wr0, b0)


def _conv1(xcat, agg1, deg, wl1, wr1, b1, wl2, wr2, b2):
    return pl.pallas_call(
        _k1_body,
        grid=(NRB,),
        in_specs=[
            pl.BlockSpec((RB, C), lambda r: (r, 0)),
            pl.BlockSpec((C // FB, RB, FB), lambda r: (0, r, 0)),
            pl.BlockSpec((RB, 16), lambda r: (r, 0)),
            pl.BlockSpec((C, C), lambda r: (0, 0)),
            pl.BlockSpec((C, C), lambda r: (0, 0)),
            pl.BlockSpec((1, C), lambda r: (0, 0)),
            pl.BlockSpec((C, OUT), lambda r: (0, 0)),
            pl.BlockSpec((C, OUT), lambda r: (0, 0)),
            pl.BlockSpec((1, OUT), lambda r: (0, 0)),
        ],
        out_specs=[
            pl.BlockSpec((RB, OUT), lambda r: (r, 0)),
            pl.BlockSpec((RB, OUT), lambda r: (r, 0)),
        ],
        out_shape=[
            jax.ShapeDtypeStruct((NPAD, OUT), F32),
            jax.ShapeDtypeStruct((NPAD, OUT), F32),
        ],
    )(xcat, agg1, deg, wl1, wr1, b1, wl2, wr2, b2)


def _final(aggy, r2, deg):
    return pl.pallas_call(
        _k2_body,
        grid=(NRB,),
        in_specs=[
            pl.BlockSpec((2, RB, FB), lambda r: (0, r, 0)),
            pl.BlockSpec((RB, OUT), lambda r: (r, 0)),
            pl.BlockSpec((RB, 16), lambda r: (r, 0)),
        ],
        out_specs=pl.BlockSpec((RB, OUT), lambda r: (r, 0)),
        out_shape=jax.ShapeDtypeStruct((NPAD, OUT), F32),
    )(aggy, r2, deg)


def kernel(x0, x1, edge_index, Wl0a, Wr0a, b0a, g0a, be0a,
           Wl0b, Wr0b, b0b, g0b, be0b, Wl1, Wr1, b1, g1, be1,
           Wl2, Wr2, b2):
    E = edge_index.shape[1]
    EPAD = NTILES * NCH * CHUNK
    src = edge_index[0]
    dst = edge_index[1]
    srcp = jnp.concatenate([src, jnp.zeros((EPAD - E,), jnp.int32)])
    dstp = jnp.concatenate([dst, jnp.full((EPAD - E,), DUMMY, jnp.int32)])
    dstp = dstp.reshape(NTILES * NCH, CHUNK)
    s2 = (srcp * 2).reshape(NTILES * NCH, CHUNK)
    s8 = (srcp * 8).reshape(NTILES * NCH, CHUNK)
    offs0 = jnp.array([0, 1, 2 * NPAD, 2 * NPAD + 1], jnp.int32)
    idx0 = s2[None] + offs0[:, None, None]
    idx1 = s8[None] + jnp.arange(8, dtype=jnp.int32)[:, None, None]
    idx2 = s2[None] + jnp.arange(2, dtype=jnp.int32)[:, None, None]

    xs = jnp.zeros((2, NPAD, D), F32).at[0, :N].set(x0).at[1, :N].set(x1)

    s0a = g0a / jnp.sqrt(1.0 + EPS)
    s0b = g0b / jnp.sqrt(1.0 + EPS)
    s1 = g1 / jnp.sqrt(1.0 + EPS)
    wl0 = jnp.stack([Wl0a * s0a, Wl0b * s0b])
    wr0 = jnp.stack([Wr0a * s0a, Wr0b * s0b])
    b0 = jnp.stack([b0a * s0a + be0a, b0b * s0b + be0b])
    wl1 = Wl1 * s1
    wr1 = Wr1 * s1
    b1f = (b1 * s1 + be1).reshape(1, C)
    b2f = b2.reshape(1, OUT)

    agg0, deg = _spmm4(xs.reshape(-1, FB), idx0, dstp)
    xcat = _conv0(agg0, xs, deg, wl0, wr0, b0)
    agg1 = _spmm8(xcat.reshape(-1, FB), idx1, dstp)
    y, r2 = _conv1(xcat, agg1, deg, wl1, wr1, b1f, Wl2, Wr2, b2f)
    aggy = _spmm2(y.reshape(-1, FB), idx2, dstp)
    out = _final(aggy, r2, deg)
    return out[:N]


# speedup vs baseline: 3.5490x; 3.5490x over previous
import functools

import jax
import jax.numpy as jnp
from jax import lax
from jax.experimental import pallas as pl
from jax.experimental.pallas import tpu as pltpu
from jax.experimental.pallas import tpu_sc as plsc

N = 10000
D = 256
H = 512
C = 2 * H
OUT = 256
EPS = 1e-5

NPAD = 10240
RB = 512
NRB = NPAD // RB
FB = 128
CHUNK = 128
NTILES = 16
NCH = 80
NH = 40
SC_ROWS = 10112
RPT = SC_ROWS // NTILES
DUMMY = N
F32 = jnp.float32


def _make_spmm(slots, with_deg):
    per = slots // 2
    mesh = plsc.VectorSubcoreMesh(core_axis_name="c", subcore_axis_name="s",
                                  num_cores=2, num_subcores=NTILES)
    out_type = [jax.ShapeDtypeStruct((slots, NPAD, FB), F32)]
    scratch = [
        pltpu.VMEM((NH, CHUNK), jnp.int32),
        pltpu.VMEM((NH, CHUNK), jnp.int32),
        pltpu.VMEM((CHUNK, FB), F32),
        pltpu.VMEM((CHUNK, FB), F32),
        pltpu.VMEM_SHARED((SC_ROWS, FB), F32),
        pltpu.SemaphoreType.DMA,
        pltpu.SemaphoreType.DMA,
    ]
    if with_deg:
        out_type.append(jax.ShapeDtypeStruct((NPAD, FB), F32))

    @functools.partial(pl.kernel, mesh=mesh, out_type=out_type,
                       scratch_types=scratch)
    def spmm(xflat, idx, dstt, zrows, *refs):
        if with_deg:
            agg_out, deg_out = refs[0], refs[1]
            gidx, dstv, r0, r1, agg_sp, sem0, sem1 = refs[2:]
        else:
            agg_out = refs[0]
            gidx, dstv, r0, r1, agg_sp, sem0, sem1 = refs[1:]
        c = lax.axis_index("c")
        t = lax.axis_index("s")
        base_r = t * RPT

        def half_pass(slot, h):
            pltpu.sync_copy(idx.at[slot, pl.ds(t * NCH + h * NH, NH)], gidx)
            pltpu.sync_copy(dstt.at[pl.ds(t * NCH + h * NH, NH)], dstv)
            pltpu.async_copy(xflat.at[gidx.at[0]], r0, sem0)

            def body(j2, _):
                j = 2 * j2
                pltpu.async_copy(xflat.at[gidx.at[j + 1]], r1, sem1)
                pltpu.make_async_copy(xflat.at[gidx.at[j]], r0, sem0).wait()
                pltpu.sync_copy(r0, agg_sp.at[dstv.at[j]], add=True)

                @pl.when(j2 < NH // 2 - 1)
                def _():
                    pltpu.async_copy(xflat.at[gidx.at[j + 2]], r0, sem0)
                pltpu.make_async_copy(xflat.at[gidx.at[j + 1]], r1, sem1).wait()
                pltpu.sync_copy(r1, agg_sp.at[dstv.at[j + 1]], add=True)
                return 0
            lax.fori_loop(0, NH // 2, body, 0)

        for k in range(per):
            slot = c * per + k
            pltpu.sync_copy(zrows, agg_sp.at[pl.ds(base_r, RPT)])
            plsc.subcore_barrier()
            for h in range(NCH // NH):
                half_pass(slot, h)
            plsc.subcore_barrier()
            pltpu.sync_copy(agg_sp.at[pl.ds(base_r, RPT)],
                            agg_out.at[slot, pl.ds(base_r, RPT)])

        if with_deg:
            @pl.when(c == 0)
            def _deg():
                def orow(i, _):
                    for l in range(FB // 16):
                        r0[i, pl.ds(l * 16, 16)] = jnp.ones((16,), F32)
                    return 0
                lax.fori_loop(0, CHUNK, orow, 0)
                pltpu.sync_copy(zrows, agg_sp.at[pl.ds(base_r, RPT)])
                plsc.subcore_barrier()

                for h in range(NCH // NH):
                    pltpu.sync_copy(dstt.at[pl.ds(t * NCH + h * NH, NH)],
                                    dstv)

                    def dchunk(j, _):
                        pltpu.sync_copy(r0, agg_sp.at[dstv.at[j]], add=True)
                        return 0
                    lax.fori_loop(0, NH, dchunk, 0)
                plsc.subcore_barrier()
                pltpu.sync_copy(agg_sp.at[pl.ds(base_r, RPT)],
                                deg_out.at[pl.ds(base_r, RPT)])

    return spmm


@functools.lru_cache(maxsize=None)
def _get_spmm(slots, with_deg):
    return _make_spmm(slots, with_deg)


def _inv_deg(deg_ref):
    return 1.0 / jnp.maximum(deg_ref[:, :1], 1.0)


def _k0_body(agg_ref, xs_ref, deg_ref, wl_ref, wr_ref, b_ref, out_ref):
    inv = _inv_deg(deg_ref)
    wl = wl_ref[0]
    acc = jnp.dot(agg_ref[0] * inv, wl[:FB], preferred_element_type=F32)
    acc += jnp.dot(agg_ref[1] * inv, wl[FB:], preferred_element_type=F32)
    acc += jnp.dot(xs_ref[0], wr_ref[0], preferred_element_type=F32)
    out_ref[...] = jnp.maximum(acc + b_ref[0], 0.0)


def _k1_body(x_ref, agg_ref, deg_ref, wl1_ref, wr1_ref, b1_ref,
             wl2_ref, wr2_ref, b2_ref, y_ref, r2_ref):
    inv = _inv_deg(deg_ref)
    acc = jnp.dot(x_ref[...], wr1_ref[...], preferred_element_type=F32)
    for fb in range(C // FB):
        acc += jnp.dot(agg_ref[fb] * inv, wl1_ref[fb * FB:(fb + 1) * FB],
                       preferred_element_type=F32)
    xn = jnp.maximum(acc + b1_ref[0][None, :], 0.0)
    y_ref[...] = jnp.dot(xn, wl2_ref[...], preferred_element_type=F32)
    r2_ref[...] = (jnp.dot(xn, wr2_ref[...], preferred_element_type=F32)
                   + b2_ref[0][None, :])


def _k2_body(aggy_ref, r2_ref, deg_ref, out_ref):
    inv = _inv_deg(deg_ref)
    out_ref[...] = (jnp.concatenate([aggy_ref[0] * inv, aggy_ref[1] * inv],
                                    axis=1) + r2_ref[...])


def _conv0(agg0, xs, deg, wl0, wr0, b0):
    return pl.pallas_call(
        _k0_body,
        grid=(2, NRB),
        in_specs=[
            pl.BlockSpec((2, RB, FB), lambda s, r: (s, r, 0)),
            pl.BlockSpec((1, RB, D), lambda s, r: (s, r, 0)),
            pl.BlockSpec((RB, FB), lambda s, r: (r, 0)),
            pl.BlockSpec((1, D, H), lambda s, r: (s, 0, 0)),
            pl.BlockSpec((1, D, H), lambda s, r: (s, 0, 0)),
            pl.BlockSpec((1, 1, H), lambda s, r: (s, 0, 0)),
        ],
        out_specs=pl.BlockSpec((RB, H), lambda s, r: (r, s)),
        out_shape=jax.ShapeDtypeStruct((NPAD, C), F32),
    )(agg0, xs, deg, wl0, wr0, b0)


def _conv1(xcat, agg1, deg, wl1, wr1, b1, wl2, wr2, b2):
    return pl.pallas_call(
        _k1_body,
        grid=(NRB,),
        in_specs=[
            pl.BlockSpec((RB, C), lambda r: (r, 0)),
            pl.BlockSpec((C // FB, RB, FB), lambda r: (0, r, 0)),
            pl.BlockSpec((RB, FB), lambda r: (r, 0)),
            pl.BlockSpec((C, C), lambda r: (0, 0)),
            pl.BlockSpec((C, C), lambda r: (0, 0)),
            pl.BlockSpec((1, C), lambda r: (0, 0)),
            pl.BlockSpec((C, OUT), lambda r: (0, 0)),
            pl.BlockSpec((C, OUT), lambda r: (0, 0)),
            pl.BlockSpec((1, OUT), lambda r: (0, 0)),
        ],
        out_specs=[
            pl.BlockSpec((RB, OUT), lambda r: (r, 0)),
            pl.BlockSpec((RB, OUT), lambda r: (r, 0)),
        ],
        out_shape=[
            jax.ShapeDtypeStruct((NPAD, OUT), F32),
            jax.ShapeDtypeStruct((NPAD, OUT), F32),
        ],
    )(xcat, agg1, deg, wl1, wr1, b1, wl2, wr2, b2)


def _final(aggy, r2, deg):
    return pl.pallas_call(
        _k2_body,
        grid=(NRB,),
        in_specs=[
            pl.BlockSpec((2, RB, FB), lambda r: (0, r, 0)),
            pl.BlockSpec((RB, OUT), lambda r: (r, 0)),
            pl.BlockSpec((RB, FB), lambda r: (r, 0)),
        ],
        out_specs=pl.BlockSpec((RB, OUT), lambda r: (r, 0)),
        out_shape=jax.ShapeDtypeStruct((NPAD, OUT), F32),
    )(aggy, r2, deg)


def kernel(x0, x1, edge_index, Wl0a, Wr0a, b0a, g0a, be0a,
           Wl0b, Wr0b, b0b, g0b, be0b, Wl1, Wr1, b1, g1, be1,
           Wl2, Wr2, b2):
    E = edge_index.shape[1]
    EPAD = NTILES * NCH * CHUNK
    src = edge_index[0]
    dst = edge_index[1]
    srcp = jnp.concatenate([src, jnp.zeros((EPAD - E,), jnp.int32)])
    dstp = jnp.concatenate([dst, jnp.full((EPAD - E,), DUMMY, jnp.int32)])
    dstp = dstp.reshape(NTILES * NCH, CHUNK)
    s2 = (srcp * 2).reshape(NTILES * NCH, CHUNK)
    s8 = (srcp * 8).reshape(NTILES * NCH, CHUNK)
    offs0 = jnp.array([0, 1, 2 * NPAD, 2 * NPAD + 1], jnp.int32)
    idx0 = s2[None] + offs0[:, None, None]
    idx1 = s8[None] + jnp.arange(8, dtype=jnp.int32)[:, None, None]
    idx2 = s2[None] + jnp.arange(2, dtype=jnp.int32)[:, None, None]

    xs = jnp.zeros((2, NPAD, D), F32).at[0, :N].set(x0).at[1, :N].set(x1)

    s0a = g0a / jnp.sqrt(1.0 + EPS)
    s0b = g0b / jnp.sqrt(1.0 + EPS)
    s1 = g1 / jnp.sqrt(1.0 + EPS)
    wl0 = jnp.stack([Wl0a * s0a, Wl0b * s0b])
    wr0 = jnp.stack([Wr0a * s0a, Wr0b * s0b])
    b0 = jnp.stack([b0a * s0a + be0a, b0b * s0b + be0b]).reshape(2, 1, H)
    wl1 = Wl1 * s1
    wr1 = Wr1 * s1
    b1f = (b1 * s1 + be1).reshape(1, C)
    b2f = b2.reshape(1, OUT)

    zrows = jnp.zeros((RPT, FB), F32)

    agg0, deg = _get_spmm(4, True)(xs.reshape(-1, FB), idx0, dstp, zrows)
    xcat = _conv0(agg0, xs, deg, wl0, wr0, b0)
    (agg1,) = _get_spmm(8, False)(xcat.reshape(-1, FB), idx1, dstp, zrows)
    y, r2 = _conv1(xcat, agg1, deg, wl1, wr1, b1f, Wl2, Wr2, b2f)
    (aggy,) = _get_spmm(2, False)(y.reshape(-1, FB), idx2, dstp, zrows)
    out = _final(aggy, r2, deg)
    return out[:N]
